# Initial kernel scaffold; baseline (speedup 1.0000x reference)
import jax
import jax.numpy as jnp
from jax.experimental import pallas as pl


def kernel(input_ids, emb, enc_rel, dec_rel, enc_ln1, enc_q, enc_k, enc_v, enc_o, enc_ln2, enc_wi, enc_wo, enc_fln, dec_ln1, dec_q, dec_k, dec_v, dec_o, dec_ln2, dec_cq, dec_ck, dec_cv, dec_co, dec_ln3, dec_router, dec_wi, dec_wo, dec_fln):
    raise NotImplementedError("write your pallas kernel here")


# speedup vs baseline: 1.0015x; 1.0015x over previous
import functools

import jax
import jax.numpy as jnp
import numpy as np
from jax.experimental import pallas as pl
from jax.experimental.pallas import tpu as pltpu

V = 32128; D = 1024; L = 2; H = 8; DK = 64; INNER = H * DK; DFF = 4096
E = 8; TOPK = 2; NB = 32; MAXD = 128
NEG = -1e30
VP = 32768
MOE_BT = 256


def _matmul(x, w, lnw=None, res=None, relu=False, bm=None, bn=512):
    M, K = x.shape
    N = w.shape[1]
    if bm is None:
        bm = 1024 if K <= 1536 else 512
    has_ln = lnw is not None
    has_res = res is not None

    def body(*refs):
        i = 0
        x_ref = refs[i]; i += 1
        if has_ln:
            ln_ref = refs[i]; i += 1
        w_ref = refs[i]; i += 1
        if has_res:
            r_ref = refs[i]; i += 1
        o_ref = refs[i]
        xv = x_ref[...]
        if has_ln:
            xv = xv * jax.lax.rsqrt(jnp.mean(xv * xv, 1, keepdims=True) + 1e-6)
            xv = xv * ln_ref[...]
        acc = jax.lax.dot_general(xv, w_ref[...], (((1,), (0,)), ((), ())),
                                  preferred_element_type=jnp.float32)
        if relu:
            acc = jnp.maximum(acc, 0.0)
        if has_res:
            acc = acc + r_ref[...]
        o_ref[...] = acc

    in_specs = [pl.BlockSpec((bm, K), lambda i, j: (i, 0))]
    ins = [x]
    if has_ln:
        in_specs.append(pl.BlockSpec((1, K), lambda i, j: (0, 0)))
        ins.append(lnw.reshape(1, K))
    in_specs.append(pl.BlockSpec((K, bn), lambda i, j: (0, j)))
    ins.append(w)
    if has_res:
        in_specs.append(pl.BlockSpec((bm, bn), lambda i, j: (i, j)))
        ins.append(res)
    return pl.pallas_call(
        body, grid=(M // bm, N // bn), in_specs=in_specs,
        out_specs=pl.BlockSpec((bm, bn), lambda i, j: (i, j)),
        out_shape=jax.ShapeDtypeStruct((M, N), jnp.float32),
    )(*ins)


def _attention(q, k, v, bias=None, causal=False, bq=256):
    B, Hh, S, dk = q.shape
    has_bias = bias is not None

    def body(*refs):
        i = 0
        q_ref = refs[i]; i += 1
        k_ref = refs[i]; i += 1
        v_ref = refs[i]; i += 1
        if has_bias:
            b_ref = refs[i]; i += 1
        o_ref = refs[i]
        qv = q_ref[0, 0]
        kv = k_ref[0, 0]
        s = jax.lax.dot_general(qv, kv, (((1,), (1,)), ((), ())),
                                preferred_element_type=jnp.float32)
        if has_bias:
            s = s + b_ref[0]
        if causal:
            rows = pl.program_id(2) * bq + jax.lax.broadcasted_iota(
                jnp.int32, (bq, S), 0)
            cols = jax.lax.broadcasted_iota(jnp.int32, (bq, S), 1)
            s = s + jnp.where(cols <= rows, 0.0, -1e9)
        m = jnp.max(s, 1, keepdims=True)
        p = jnp.exp(s - m)
        a = p / jnp.sum(p, 1, keepdims=True)
        o_ref[0, 0] = jax.lax.dot_general(a, v_ref[0, 0],
                                          (((1,), (0,)), ((), ())),
                                          preferred_element_type=jnp.float32)

    in_specs = [
        pl.BlockSpec((1, 1, bq, dk), lambda b, h, i: (b, h, i, 0)),
        pl.BlockSpec((1, 1, S, dk), lambda b, h, i: (b, h, 0, 0)),
        pl.BlockSpec((1, 1, S, dk), lambda b, h, i: (b, h, 0, 0)),
    ]
    ins = [q, k, v]
    if has_bias:
        in_specs.append(pl.BlockSpec((1, bq, S), lambda b, h, i: (h, i, 0)))
        ins.append(bias)
    return pl.pallas_call(
        body, grid=(B, Hh, S // bq), in_specs=in_specs,
        out_specs=pl.BlockSpec((1, 1, bq, dk), lambda b, h, i: (b, h, i, 0)),
        out_shape=jax.ShapeDtypeStruct((B, Hh, S, dk), jnp.float32),
    )(*ins)


def _router(h2, lnw, rw_pad, bm=512):
    T = h2.shape[0]

    def body(x_ref, ln_ref, rw_ref, xln_ref, lg_ref):
        xv = x_ref[...]
        xn = xv * jax.lax.rsqrt(jnp.mean(xv * xv, 1, keepdims=True) + 1e-6)
        xn = xn * ln_ref[...]
        xln_ref[...] = xn
        lg = jax.lax.dot_general(xn, rw_ref[...], (((1,), (0,)), ((), ())),
                                 preferred_element_type=jnp.float32)
        lanes = jax.lax.broadcasted_iota(jnp.int32, (bm, 128), 1)
        lg_ref[...] = jnp.where(lanes < E, lg, NEG)

    return pl.pallas_call(
        body, grid=(T // bm,),
        in_specs=[pl.BlockSpec((bm, D), lambda i: (i, 0)),
                  pl.BlockSpec((1, D), lambda i: (0, 0)),
                  pl.BlockSpec((D, 128), lambda i: (0, 0))],
        out_specs=(pl.BlockSpec((bm, D), lambda i: (i, 0)),
                   pl.BlockSpec((bm, 128), lambda i: (i, 0))),
        out_shape=(jax.ShapeDtypeStruct((T, D), jnp.float32),
                   jax.ShapeDtypeStruct((T, 128), jnp.float32)),
    )(h2, lnw.reshape(1, D), rw_pad)


def _moe_mm1(xg, wi, te):
    PT = xg.shape[0]

    def body(te_ref, x_ref, w_ref, o_ref):
        del te_ref
        o_ref[...] = jnp.maximum(
            jax.lax.dot_general(x_ref[...], w_ref[0], (((1,), (0,)), ((), ())),
                                preferred_element_type=jnp.float32), 0.0)

    spec = pltpu.PrefetchScalarGridSpec(
        num_scalar_prefetch=1, grid=(PT // MOE_BT,),
        in_specs=[pl.BlockSpec((MOE_BT, D), lambda t, te: (t, 0)),
                  pl.BlockSpec((1, D, DFF), lambda t, te: (te[t], 0, 0))],
        out_specs=pl.BlockSpec((MOE_BT, DFF), lambda t, te: (t, 0)),
    )
    return pl.pallas_call(
        body, grid_spec=spec,
        out_shape=jax.ShapeDtypeStruct((PT, DFF), jnp.float32))(te, xg, wi)


def _moe_mm2(hm, wo, te):
    PT = hm.shape[0]

    def body(te_ref, x_ref, w_ref, o_ref):
        del te_ref
        o_ref[...] = jax.lax.dot_general(
            x_ref[...], w_ref[0], (((1,), (0,)), ((), ())),
            preferred_element_type=jnp.float32)

    spec = pltpu.PrefetchScalarGridSpec(
        num_scalar_prefetch=1, grid=(PT // MOE_BT,),
        in_specs=[pl.BlockSpec((MOE_BT, DFF), lambda t, te: (t, 0)),
                  pl.BlockSpec((1, DFF, D), lambda t, te: (te[t], 0, 0))],
        out_specs=pl.BlockSpec((MOE_BT, D), lambda t, te: (t, 0)),
    )
    return pl.pallas_call(
        body, grid_spec=spec,
        out_shape=jax.ShapeDtypeStruct((PT, D), jnp.float32))(te, hm, wo)


def _moe_dispatch(lg, T):
    probs = jax.nn.softmax(lg[:, :E], -1)
    topv, topi = jax.lax.top_k(probs, TOPK)
    gates = topv / jnp.sum(topv, -1, keepdims=True)
    F = TOPK * T
    eflat = topi.reshape(F)
    gflat = gates.reshape(F)
    tok = jnp.arange(F, dtype=jnp.int32) // TOPK
    order = jnp.argsort(eflat)
    es = eflat[order]
    ts = tok[order]
    sizes = jnp.bincount(eflat, length=E).astype(jnp.int32)
    off = jnp.concatenate([jnp.zeros(1, jnp.int32), jnp.cumsum(sizes)[:-1]])
    ps = ((sizes + MOE_BT - 1) // MOE_BT) * MOE_BT
    poff = jnp.concatenate([jnp.zeros(1, jnp.int32), jnp.cumsum(ps)[:-1]])
    ppos = jnp.arange(F, dtype=jnp.int32) - off[es] + poff[es]
    PT = F + E * MOE_BT
    tokp = jnp.zeros(PT, jnp.int32).at[ppos].set(ts)
    tiles = jnp.arange(PT // MOE_BT, dtype=jnp.int32) * MOE_BT
    te = (jnp.searchsorted(poff, tiles, side='right') - 1).astype(jnp.int32)
    te = jnp.clip(te, 0, E - 1)
    ipos = jnp.zeros(F, jnp.int32).at[order].set(ppos)
    return tokp, te, ipos, gflat


def _lm_loss(h2, fln, emb_pad, labels2d, bt=1024, bv=2048):
    T = h2.shape[0]
    nv = VP // bv
    scale = float(D) ** -0.5

    def body(h_ref, ln_ref, e_ref, lbl_ref, o_ref, m_s, s_s, l_s):
        vc = pl.program_id(1)
        hv = h_ref[...]
        hn = hv * jax.lax.rsqrt(jnp.mean(hv * hv, 1, keepdims=True) + 1e-6)
        hn = hn * ln_ref[...] * scale
        lg = jax.lax.dot_general(hn, e_ref[...], (((1,), (1,)), ((), ())),
                                 preferred_element_type=jnp.float32)
        cols = vc * bv + jax.lax.broadcasted_iota(jnp.int32, (bt, bv), 1)
        lg = jnp.where(cols < V, lg, NEG)
        lblv = lbl_ref[:, 0:1]
        lhit = jnp.sum(jnp.where(cols == lblv, lg, 0.0), 1, keepdims=True)
        mc = jnp.max(lg, 1, keepdims=True)

        @pl.when(vc == 0)
        def _init():
            m_s[...] = jnp.broadcast_to(mc, (bt, 128))
            s_s[...] = jnp.broadcast_to(
                jnp.sum(jnp.exp(lg - mc), 1, keepdims=True), (bt, 128))
            l_s[...] = jnp.broadcast_to(lhit, (bt, 128))

        @pl.when(vc > 0)
        def _update():
            m_old = jnp.max(m_s[...], 1, keepdims=True)
            s_old = jnp.max(s_s[...], 1, keepdims=True)
            m_new = jnp.maximum(m_old, mc)
            s_new = (s_old * jnp.exp(m_old - m_new)
                     + jnp.sum(jnp.exp(lg - m_new), 1, keepdims=True))
            m_s[...] = jnp.broadcast_to(m_new, (bt, 128))
            s_s[...] = jnp.broadcast_to(s_new, (bt, 128))
            l_s[...] = l_s[...] + lhit

        @pl.when(vc == nv - 1)
        def _finish():
            o_ref[...] = m_s[...] + jnp.log(s_s[...]) - l_s[...]

    return pl.pallas_call(
        body, grid=(T // bt, nv),
        in_specs=[pl.BlockSpec((bt, D), lambda t, v: (t, 0)),
                  pl.BlockSpec((1, D), lambda t, v: (0, 0)),
                  pl.BlockSpec((bv, D), lambda t, v: (v, 0)),
                  pl.BlockSpec((bt, 128), lambda t, v: (t, 0))],
        out_specs=pl.BlockSpec((bt, 128), lambda t, v: (t, 0)),
        out_shape=jax.ShapeDtypeStruct((T, 128), jnp.float32),
        scratch_shapes=[pltpu.VMEM((bt, 128), jnp.float32)] * 3,
    )(h2, fln.reshape(1, D), emb_pad, labels2d)


def _bucket(relpos, bidirectional):
    if bidirectional:
        nb = NB // 2
        rb = (relpos > 0).astype(jnp.int32) * nb
        rp = jnp.abs(relpos)
    else:
        nb = NB
        rb = jnp.zeros_like(relpos)
        rp = -jnp.minimum(relpos, 0)
    max_exact = nb // 2
    is_small = rp < max_exact
    large = max_exact + (jnp.log(rp.astype(jnp.float32) / max_exact + 1e-9)
                         / np.log(MAXD / max_exact)
                         * (nb - max_exact)).astype(jnp.int32)
    large = jnp.minimum(large, nb - 1)
    return rb + jnp.where(is_small, rp, large)


def _mk_bias(table, S, bidirectional):
    rel = jnp.arange(S)[None, :] - jnp.arange(S)[:, None]
    return table[_bucket(rel, bidirectional)].transpose(2, 0, 1)


def _heads(t, B, S):
    return t.reshape(B, S, H, DK).transpose(0, 2, 1, 3)


def _unheads(t, T):
    return t.transpose(0, 2, 1, 3).reshape(T, INNER)


def kernel(input_ids, emb, enc_rel, dec_rel, enc_ln1, enc_q, enc_k, enc_v,
           enc_o, enc_ln2, enc_wi, enc_wo, enc_fln, dec_ln1, dec_q, dec_k,
           dec_v, dec_o, dec_ln2, dec_cq, dec_ck, dec_cv, dec_co, dec_ln3,
           dec_router, dec_wi, dec_wo, dec_fln):
    B, S = input_ids.shape
    T = B * S
    labels = input_ids.reshape(T)
    dec_ids = jnp.concatenate(
        [jnp.zeros_like(input_ids[:, :1]), input_ids[:, :-1]], 1)
    enc_bias = _mk_bias(enc_rel, S, True)
    dec_bias = _mk_bias(dec_rel, S, False)

    h = emb[input_ids].reshape(T, D)
    for i in range(L):
        qkv_w = jnp.concatenate([enc_q[i], enc_k[i], enc_v[i]], axis=1)
        qkv = _matmul(h, qkv_w, lnw=enc_ln1[i])
        q, k, v = (qkv[:, :INNER], qkv[:, INNER:2 * INNER], qkv[:, 2 * INNER:])
        ao = _attention(_heads(q, B, S), _heads(k, B, S), _heads(v, B, S),
                        bias=enc_bias)
        h = _matmul(_unheads(ao, T), enc_o[i], res=h)
        t1 = _matmul(h, enc_wi[i], lnw=enc_ln2[i], relu=True)
        h = _matmul(t1, enc_wo[i], res=h)
    h_enc = h

    hd = emb[dec_ids].reshape(T, D)
    for i in range(L):
        qkv_w = jnp.concatenate([dec_q[i], dec_k[i], dec_v[i]], axis=1)
        qkv = _matmul(hd, qkv_w, lnw=dec_ln1[i])
        q, k, v = (qkv[:, :INNER], qkv[:, INNER:2 * INNER], qkv[:, 2 * INNER:])
        ao = _attention(_heads(q, B, S), _heads(k, B, S), _heads(v, B, S),
                        bias=dec_bias, causal=True)
        hd = _matmul(_unheads(ao, T), dec_o[i], res=hd)

        cq = _matmul(hd, dec_cq[i], lnw=dec_ln2[i])
        ckv_w = jnp.concatenate([dec_ck[i], dec_cv[i]], axis=1)
        ckv = _matmul(h_enc, ckv_w, lnw=enc_fln)
        ck, cv = ckv[:, :INNER], ckv[:, INNER:]
        co = _attention(_heads(cq, B, S), _heads(ck, B, S), _heads(cv, B, S))
        hd = _matmul(_unheads(co, T), dec_co[i], res=hd)

        rw_pad = jnp.pad(dec_router[i], ((0, 0), (0, 128 - E)))
        xln, lg = _router(hd, dec_ln3[i], rw_pad)
        tokp, te, ipos, gflat = _moe_dispatch(lg, T)
        xg = xln[tokp]
        hm = _moe_mm1(xg, dec_wi[i], te)
        ys = _moe_mm2(hm, dec_wo[i], te)
        y = (ys[ipos] * gflat[:, None]).reshape(T, TOPK, D).sum(1)
        hd = hd + y

    emb_pad = jnp.pad(emb, ((0, VP - V), (0, 0)))
    labels2d = jnp.broadcast_to(labels[:, None], (T, 128)).astype(jnp.int32)
    nll = _lm_loss(hd, dec_fln, emb_pad, labels2d)
    return jnp.mean(nll[:, 0])


# speedup vs baseline: 1.0090x; 1.0075x over previous
import functools

import jax
import jax.numpy as jnp
import numpy as np
from jax.experimental import pallas as pl
from jax.experimental.pallas import tpu as pltpu

V = 32128; D = 1024; L = 2; H = 8; DK = 64; INNER = H * DK; DFF = 4096
E = 8; TOPK = 2; NB = 32; MAXD = 128
NEG = -1e30
VP = 32768
MOE_BT = 256


def _matmul(x, w, lnw=None, res=None, relu=False, bm=None, bn=512,
            out_dtype=jnp.float32):
    M, K = x.shape
    N = w.shape[1]
    if bm is None:
        bm = 1024 if K <= 1536 else 512
    has_ln = lnw is not None
    has_res = res is not None
    w = w.astype(jnp.bfloat16)

    def body(*refs):
        i = 0
        x_ref = refs[i]; i += 1
        if has_ln:
            ln_ref = refs[i]; i += 1
        w_ref = refs[i]; i += 1
        if has_res:
            r_ref = refs[i]; i += 1
        o_ref = refs[i]
        xv = x_ref[...]
        if has_ln:
            xf = xv.astype(jnp.float32)
            xf = xf * jax.lax.rsqrt(jnp.mean(xf * xf, 1, keepdims=True) + 1e-6)
            xv = xf * ln_ref[...]
        acc = jax.lax.dot_general(xv.astype(jnp.bfloat16), w_ref[...],
                                  (((1,), (0,)), ((), ())),
                                  preferred_element_type=jnp.float32)
        if relu:
            acc = jnp.maximum(acc, 0.0)
        if has_res:
            acc = acc + r_ref[...]
        o_ref[...] = acc.astype(o_ref.dtype)

    in_specs = [pl.BlockSpec((bm, K), lambda i, j: (i, 0))]
    ins = [x]
    if has_ln:
        in_specs.append(pl.BlockSpec((1, K), lambda i, j: (0, 0)))
        ins.append(lnw.reshape(1, K))
    in_specs.append(pl.BlockSpec((K, bn), lambda i, j: (0, j)))
    ins.append(w)
    if has_res:
        in_specs.append(pl.BlockSpec((bm, bn), lambda i, j: (i, j)))
        ins.append(res)
    return pl.pallas_call(
        body, grid=(M // bm, N // bn), in_specs=in_specs,
        out_specs=pl.BlockSpec((bm, bn), lambda i, j: (i, j)),
        out_shape=jax.ShapeDtypeStruct((M, N), out_dtype),
    )(*ins)


def _attention(q, k, v, bias=None, causal=False, bq=256):
    B, Hh, S, dk = q.shape
    has_bias = bias is not None

    def body(*refs):
        i = 0
        q_ref = refs[i]; i += 1
        k_ref = refs[i]; i += 1
        v_ref = refs[i]; i += 1
        if has_bias:
            b_ref = refs[i]; i += 1
        o_ref = refs[i]
        qv = q_ref[0, 0].astype(jnp.bfloat16)
        kv = k_ref[0, 0].astype(jnp.bfloat16)
        s = jax.lax.dot_general(qv, kv, (((1,), (1,)), ((), ())),
                                preferred_element_type=jnp.float32)
        if has_bias:
            s = s + b_ref[0]
        if causal:
            rows = pl.program_id(2) * bq + jax.lax.broadcasted_iota(
                jnp.int32, (bq, S), 0)
            cols = jax.lax.broadcasted_iota(jnp.int32, (bq, S), 1)
            s = s + jnp.where(cols <= rows, 0.0, -1e9)
        m = jnp.max(s, 1, keepdims=True)
        p = jnp.exp(s - m)
        a = (p / jnp.sum(p, 1, keepdims=True)).astype(jnp.bfloat16)
        o_ref[0, 0] = jax.lax.dot_general(a, v_ref[0, 0].astype(jnp.bfloat16),
                                          (((1,), (0,)), ((), ())),
                                          preferred_element_type=jnp.float32)

    in_specs = [
        pl.BlockSpec((1, 1, bq, dk), lambda b, h, i: (b, h, i, 0)),
        pl.BlockSpec((1, 1, S, dk), lambda b, h, i: (b, h, 0, 0)),
        pl.BlockSpec((1, 1, S, dk), lambda b, h, i: (b, h, 0, 0)),
    ]
    ins = [q, k, v]
    if has_bias:
        in_specs.append(pl.BlockSpec((1, bq, S), lambda b, h, i: (h, i, 0)))
        ins.append(bias)
    return pl.pallas_call(
        body, grid=(B, Hh, S // bq), in_specs=in_specs,
        out_specs=pl.BlockSpec((1, 1, bq, dk), lambda b, h, i: (b, h, i, 0)),
        out_shape=jax.ShapeDtypeStruct((B, Hh, S, dk), jnp.float32),
    )(*ins)


def _router(h2, lnw, rw_pad, bm=512):
    T = h2.shape[0]

    def body(x_ref, ln_ref, rw_ref, xln_ref, lg_ref):
        xv = x_ref[...]
        xn = xv * jax.lax.rsqrt(jnp.mean(xv * xv, 1, keepdims=True) + 1e-6)
        xn = xn * ln_ref[...]
        xln_ref[...] = xn
        lg = jax.lax.dot_general(xn, rw_ref[...], (((1,), (0,)), ((), ())),
                                 preferred_element_type=jnp.float32)
        lanes = jax.lax.broadcasted_iota(jnp.int32, (bm, 128), 1)
        lg_ref[...] = jnp.where(lanes < E, lg, NEG)

    return pl.pallas_call(
        body, grid=(T // bm,),
        in_specs=[pl.BlockSpec((bm, D), lambda i: (i, 0)),
                  pl.BlockSpec((1, D), lambda i: (0, 0)),
                  pl.BlockSpec((D, 128), lambda i: (0, 0))],
        out_specs=(pl.BlockSpec((bm, D), lambda i: (i, 0)),
                   pl.BlockSpec((bm, 128), lambda i: (i, 0))),
        out_shape=(jax.ShapeDtypeStruct((T, D), jnp.float32),
                   jax.ShapeDtypeStruct((T, 128), jnp.float32)),
    )(h2, lnw.reshape(1, D), rw_pad)


def _moe_mm1(xg, wi, te):
    PT = xg.shape[0]

    def body(te_ref, x_ref, w_ref, o_ref):
        del te_ref
        acc = jax.lax.dot_general(x_ref[...].astype(jnp.bfloat16), w_ref[0],
                                  (((1,), (0,)), ((), ())),
                                  preferred_element_type=jnp.float32)
        o_ref[...] = jnp.maximum(acc, 0.0).astype(jnp.bfloat16)

    spec = pltpu.PrefetchScalarGridSpec(
        num_scalar_prefetch=1, grid=(PT // MOE_BT,),
        in_specs=[pl.BlockSpec((MOE_BT, D), lambda t, te: (t, 0)),
                  pl.BlockSpec((1, D, DFF), lambda t, te: (te[t], 0, 0))],
        out_specs=pl.BlockSpec((MOE_BT, DFF), lambda t, te: (t, 0)),
    )
    return pl.pallas_call(
        body, grid_spec=spec,
        out_shape=jax.ShapeDtypeStruct((PT, DFF), jnp.bfloat16))(
            te, xg, wi.astype(jnp.bfloat16))


def _moe_mm2(hm, wo, te):
    PT = hm.shape[0]

    def body(te_ref, x_ref, w_ref, o_ref):
        del te_ref
        o_ref[...] = jax.lax.dot_general(
            x_ref[...], w_ref[0], (((1,), (0,)), ((), ())),
            preferred_element_type=jnp.float32)

    spec = pltpu.PrefetchScalarGridSpec(
        num_scalar_prefetch=1, grid=(PT // MOE_BT,),
        in_specs=[pl.BlockSpec((MOE_BT, DFF), lambda t, te: (t, 0)),
                  pl.BlockSpec((1, DFF, D), lambda t, te: (te[t], 0, 0))],
        out_specs=pl.BlockSpec((MOE_BT, D), lambda t, te: (t, 0)),
    )
    return pl.pallas_call(
        body, grid_spec=spec,
        out_shape=jax.ShapeDtypeStruct((PT, D), jnp.float32))(
            te, hm, wo.astype(jnp.bfloat16))


def _moe_dispatch(lg, T):
    probs = jax.nn.softmax(lg[:, :E], -1)
    topv, topi = jax.lax.top_k(probs, TOPK)
    gates = topv / jnp.sum(topv, -1, keepdims=True)
    F = TOPK * T
    eflat = topi.reshape(F)
    gflat = gates.reshape(F)
    tok = jnp.arange(F, dtype=jnp.int32) // TOPK
    order = jnp.argsort(eflat)
    es = eflat[order]
    ts = tok[order]
    sizes = jnp.bincount(eflat, length=E).astype(jnp.int32)
    off = jnp.concatenate([jnp.zeros(1, jnp.int32), jnp.cumsum(sizes)[:-1]])
    ps = ((sizes + MOE_BT - 1) // MOE_BT) * MOE_BT
    poff = jnp.concatenate([jnp.zeros(1, jnp.int32), jnp.cumsum(ps)[:-1]])
    ppos = jnp.arange(F, dtype=jnp.int32) - off[es] + poff[es]
    PT = F + E * MOE_BT
    tokp = jnp.zeros(PT, jnp.int32).at[ppos].set(ts)
    tiles = jnp.arange(PT // MOE_BT, dtype=jnp.int32) * MOE_BT
    te = (jnp.searchsorted(poff, tiles, side='right') - 1).astype(jnp.int32)
    te = jnp.clip(te, 0, E - 1)
    ipos = jnp.zeros(F, jnp.int32).at[order].set(ppos)
    return tokp, te, ipos, gflat


def _lm_loss(h2, fln, emb_pad, labels2d, bt=1024, bv=2048):
    T = h2.shape[0]
    nv = VP // bv
    scale = float(D) ** -0.5

    def body(h_ref, ln_ref, e_ref, lbl_ref, o_ref, m_s, s_s, l_s):
        vc = pl.program_id(1)
        hv = h_ref[...]
        hn = hv * jax.lax.rsqrt(jnp.mean(hv * hv, 1, keepdims=True) + 1e-6)
        hn = (hn * ln_ref[...] * scale).astype(jnp.bfloat16)
        lg = jax.lax.dot_general(hn, e_ref[...], (((1,), (1,)), ((), ())),
                                 preferred_element_type=jnp.float32)
        cols = vc * bv + jax.lax.broadcasted_iota(jnp.int32, (bt, bv), 1)
        lg = jnp.where(cols < V, lg, NEG)
        lblv = lbl_ref[:, 0:1]
        lhit = jnp.sum(jnp.where(cols == lblv, lg, 0.0), 1, keepdims=True)
        mc = jnp.max(lg, 1, keepdims=True)

        @pl.when(vc == 0)
        def _init():
            m_s[...] = jnp.broadcast_to(mc, (bt, 128))
            s_s[...] = jnp.broadcast_to(
                jnp.sum(jnp.exp(lg - mc), 1, keepdims=True), (bt, 128))
            l_s[...] = jnp.broadcast_to(lhit, (bt, 128))

        @pl.when(vc > 0)
        def _update():
            m_old = jnp.max(m_s[...], 1, keepdims=True)
            s_old = jnp.max(s_s[...], 1, keepdims=True)
            m_new = jnp.maximum(m_old, mc)
            s_new = (s_old * jnp.exp(m_old - m_new)
                     + jnp.sum(jnp.exp(lg - m_new), 1, keepdims=True))
            m_s[...] = jnp.broadcast_to(m_new, (bt, 128))
            s_s[...] = jnp.broadcast_to(s_new, (bt, 128))
            l_s[...] = l_s[...] + lhit

        @pl.when(vc == nv - 1)
        def _finish():
            o_ref[...] = m_s[...] + jnp.log(s_s[...]) - l_s[...]

    return pl.pallas_call(
        body, grid=(T // bt, nv),
        in_specs=[pl.BlockSpec((bt, D), lambda t, v: (t, 0)),
                  pl.BlockSpec((1, D), lambda t, v: (0, 0)),
                  pl.BlockSpec((bv, D), lambda t, v: (v, 0)),
                  pl.BlockSpec((bt, 128), lambda t, v: (t, 0))],
        out_specs=pl.BlockSpec((bt, 128), lambda t, v: (t, 0)),
        out_shape=jax.ShapeDtypeStruct((T, 128), jnp.float32),
        scratch_shapes=[pltpu.VMEM((bt, 128), jnp.float32)] * 3,
    )(h2, fln.reshape(1, D), emb_pad, labels2d)


def _bucket(relpos, bidirectional):
    if bidirectional:
        nb = NB // 2
        rb = (relpos > 0).astype(jnp.int32) * nb
        rp = jnp.abs(relpos)
    else:
        nb = NB
        rb = jnp.zeros_like(relpos)
        rp = -jnp.minimum(relpos, 0)
    max_exact = nb // 2
    is_small = rp < max_exact
    large = max_exact + (jnp.log(rp.astype(jnp.float32) / max_exact + 1e-9)
                         / np.log(MAXD / max_exact)
                         * (nb - max_exact)).astype(jnp.int32)
    large = jnp.minimum(large, nb - 1)
    return rb + jnp.where(is_small, rp, large)


def _mk_bias(table, S, bidirectional):
    rel = jnp.arange(S)[None, :] - jnp.arange(S)[:, None]
    return table[_bucket(rel, bidirectional)].transpose(2, 0, 1)


def _heads(t, B, S):
    return t.reshape(B, S, H, DK).transpose(0, 2, 1, 3)


def _unheads(t, T):
    return t.transpose(0, 2, 1, 3).reshape(T, INNER)


def kernel(input_ids, emb, enc_rel, dec_rel, enc_ln1, enc_q, enc_k, enc_v,
           enc_o, enc_ln2, enc_wi, enc_wo, enc_fln, dec_ln1, dec_q, dec_k,
           dec_v, dec_o, dec_ln2, dec_cq, dec_ck, dec_cv, dec_co, dec_ln3,
           dec_router, dec_wi, dec_wo, dec_fln):
    B, S = input_ids.shape
    T = B * S
    labels = input_ids.reshape(T)
    dec_ids = jnp.concatenate(
        [jnp.zeros_like(input_ids[:, :1]), input_ids[:, :-1]], 1)
    enc_bias = _mk_bias(enc_rel, S, True)
    dec_bias = _mk_bias(dec_rel, S, False)

    h = emb[input_ids].reshape(T, D)
    for i in range(L):
        qkv_w = jnp.concatenate([enc_q[i], enc_k[i], enc_v[i]], axis=1)
        qkv = _matmul(h, qkv_w, lnw=enc_ln1[i])
        q, k, v = (qkv[:, :INNER], qkv[:, INNER:2 * INNER], qkv[:, 2 * INNER:])
        ao = _attention(_heads(q, B, S), _heads(k, B, S), _heads(v, B, S),
                        bias=enc_bias)
        h = _matmul(_unheads(ao, T), enc_o[i], res=h)
        t1 = _matmul(h, enc_wi[i], lnw=enc_ln2[i], relu=True,
                     out_dtype=jnp.bfloat16)
        h = _matmul(t1, enc_wo[i], res=h)
    h_enc = h

    hd = emb[dec_ids].reshape(T, D)
    for i in range(L):
        qkv_w = jnp.concatenate([dec_q[i], dec_k[i], dec_v[i]], axis=1)
        qkv = _matmul(hd, qkv_w, lnw=dec_ln1[i])
        q, k, v = (qkv[:, :INNER], qkv[:, INNER:2 * INNER], qkv[:, 2 * INNER:])
        ao = _attention(_heads(q, B, S), _heads(k, B, S), _heads(v, B, S),
                        bias=dec_bias, causal=True)
        hd = _matmul(_unheads(ao, T), dec_o[i], res=hd)

        cq = _matmul(hd, dec_cq[i], lnw=dec_ln2[i])
        ckv_w = jnp.concatenate([dec_ck[i], dec_cv[i]], axis=1)
        ckv = _matmul(h_enc, ckv_w, lnw=enc_fln)
        ck, cv = ckv[:, :INNER], ckv[:, INNER:]
        co = _attention(_heads(cq, B, S), _heads(ck, B, S), _heads(cv, B, S))
        hd = _matmul(_unheads(co, T), dec_co[i], res=hd)

        rw_pad = jnp.pad(dec_router[i], ((0, 0), (0, 128 - E)))
        xln, lg = _router(hd, dec_ln3[i], rw_pad)
        tokp, te, ipos, gflat = _moe_dispatch(lg, T)
        xg = xln[tokp]
        hm = _moe_mm1(xg, dec_wi[i], te)
        ys = _moe_mm2(hm, dec_wo[i], te)
        y = (ys[ipos] * gflat[:, None]).reshape(T, TOPK, D).sum(1)
        hd = hd + y

    emb_pad = jnp.pad(emb, ((0, VP - V), (0, 0))).astype(jnp.bfloat16)
    labels2d = jnp.broadcast_to(labels[:, None], (T, 128)).astype(jnp.int32)
    nll = _lm_loss(hd, dec_fln, emb_pad, labels2d)
    return jnp.mean(nll[:, 0])


# speedup vs baseline: 6.5417x; 6.4831x over previous
import functools

import jax
import jax.numpy as jnp
import numpy as np
from jax.experimental import pallas as pl
from jax.experimental.pallas import tpu as pltpu

V = 32128; D = 1024; L = 2; H = 8; DK = 64; INNER = H * DK; DFF = 4096
E = 8; TOPK = 2; NB = 32; MAXD = 128
NEG = -1e30
VP = 32768
MOE_BT = 256


def _matmul(x, w, lnw=None, res=None, relu=False, bm=None, bn=512,
            out_dtype=jnp.float32):
    M, K = x.shape
    N = w.shape[1]
    if bm is None:
        bm = 1024 if K <= 1536 else 512
    has_ln = lnw is not None
    has_res = res is not None
    w = w.astype(jnp.bfloat16)

    def body(*refs):
        i = 0
        x_ref = refs[i]; i += 1
        if has_ln:
            ln_ref = refs[i]; i += 1
        w_ref = refs[i]; i += 1
        if has_res:
            r_ref = refs[i]; i += 1
        o_ref = refs[i]
        xv = x_ref[...]
        if has_ln:
            xf = xv.astype(jnp.float32)
            xf = xf * jax.lax.rsqrt(jnp.mean(xf * xf, 1, keepdims=True) + 1e-6)
            xv = xf * ln_ref[...]
        acc = jax.lax.dot_general(xv.astype(jnp.bfloat16), w_ref[...],
                                  (((1,), (0,)), ((), ())),
                                  preferred_element_type=jnp.float32)
        if relu:
            acc = jnp.maximum(acc, 0.0)
        if has_res:
            acc = acc + r_ref[...]
        o_ref[...] = acc.astype(o_ref.dtype)

    in_specs = [pl.BlockSpec((bm, K), lambda i, j: (i, 0))]
    ins = [x]
    if has_ln:
        in_specs.append(pl.BlockSpec((1, K), lambda i, j: (0, 0)))
        ins.append(lnw.reshape(1, K))
    in_specs.append(pl.BlockSpec((K, bn), lambda i, j: (0, j)))
    ins.append(w)
    if has_res:
        in_specs.append(pl.BlockSpec((bm, bn), lambda i, j: (i, j)))
        ins.append(res)
    return pl.pallas_call(
        body, grid=(M // bm, N // bn), in_specs=in_specs,
        out_specs=pl.BlockSpec((bm, bn), lambda i, j: (i, j)),
        out_shape=jax.ShapeDtypeStruct((M, N), out_dtype),
    )(*ins)


def _attention(q, k, v, bias_win=None, causal=False, bq=256):
    B, Hh, S, dk = q.shape
    has_bias = bias_win is not None
    QT = S // bq
    SW = S + bq

    def body(*refs):
        i = 0
        q_ref = refs[i]; i += 1
        k_ref = refs[i]; i += 1
        v_ref = refs[i]; i += 1
        if has_bias:
            b_ref = refs[i]; i += 1
        o_ref = refs[i]
        qv = q_ref[0, 0].astype(jnp.bfloat16)
        kv = k_ref[0, 0].astype(jnp.bfloat16)
        s = jax.lax.dot_general(qv, kv, (((1,), (1,)), ((), ())),
                                preferred_element_type=jnp.float32)
        if has_bias:
            ri = jax.lax.broadcasted_iota(jnp.int32, (bq, SW), 0)
            srow = (bq - 1) - ri
            X = jnp.broadcast_to(b_ref[0], (bq, SW))
            kk = 1
            while kk < bq:
                rolled = jnp.concatenate([X[:, kk:], X[:, :kk]], axis=1)
                X = jnp.where((srow & kk) != 0, rolled, X)
                kk *= 2
            s = s + X[:, :S]
        if causal:
            rows = pl.program_id(2) * bq + jax.lax.broadcasted_iota(
                jnp.int32, (bq, S), 0)
            cols = jax.lax.broadcasted_iota(jnp.int32, (bq, S), 1)
            s = s + jnp.where(cols <= rows, 0.0, -1e9)
        m = jnp.max(s, 1, keepdims=True)
        p = jnp.exp(s - m)
        a = (p / jnp.sum(p, 1, keepdims=True)).astype(jnp.bfloat16)
        o_ref[0, 0] = jax.lax.dot_general(a, v_ref[0, 0].astype(jnp.bfloat16),
                                          (((1,), (0,)), ((), ())),
                                          preferred_element_type=jnp.float32)

    in_specs = [
        pl.BlockSpec((1, 1, bq, dk), lambda b, h, i: (b, h, i, 0)),
        pl.BlockSpec((1, 1, S, dk), lambda b, h, i: (b, h, 0, 0)),
        pl.BlockSpec((1, 1, S, dk), lambda b, h, i: (b, h, 0, 0)),
    ]
    ins = [q, k, v]
    if has_bias:
        in_specs.append(
            pl.BlockSpec((1, 1, SW), lambda b, h, i: (h * QT + i, 0, 0)))
        ins.append(bias_win)
    return pl.pallas_call(
        body, grid=(B, Hh, S // bq), in_specs=in_specs,
        out_specs=pl.BlockSpec((1, 1, bq, dk), lambda b, h, i: (b, h, i, 0)),
        out_shape=jax.ShapeDtypeStruct((B, Hh, S, dk), jnp.float32),
    )(*ins)


def _router(h2, lnw, rw_pad, bm=512):
    T = h2.shape[0]

    def body(x_ref, ln_ref, rw_ref, xln_ref, lg_ref):
        xv = x_ref[...]
        xn = xv * jax.lax.rsqrt(jnp.mean(xv * xv, 1, keepdims=True) + 1e-6)
        xn = xn * ln_ref[...]
        xln_ref[...] = xn
        lg = jax.lax.dot_general(xn, rw_ref[...], (((1,), (0,)), ((), ())),
                                 preferred_element_type=jnp.float32)
        lanes = jax.lax.broadcasted_iota(jnp.int32, (bm, 128), 1)
        lg_ref[...] = jnp.where(lanes < E, lg, NEG)

    return pl.pallas_call(
        body, grid=(T // bm,),
        in_specs=[pl.BlockSpec((bm, D), lambda i: (i, 0)),
                  pl.BlockSpec((1, D), lambda i: (0, 0)),
                  pl.BlockSpec((D, 128), lambda i: (0, 0))],
        out_specs=(pl.BlockSpec((bm, D), lambda i: (i, 0)),
                   pl.BlockSpec((bm, 128), lambda i: (i, 0))),
        out_shape=(jax.ShapeDtypeStruct((T, D), jnp.float32),
                   jax.ShapeDtypeStruct((T, 128), jnp.float32)),
    )(h2, lnw.reshape(1, D), rw_pad)


def _moe_mm1(xg, wi, te):
    PT = xg.shape[0]

    def body(te_ref, x_ref, w_ref, o_ref):
        del te_ref
        acc = jax.lax.dot_general(x_ref[...].astype(jnp.bfloat16), w_ref[0],
                                  (((1,), (0,)), ((), ())),
                                  preferred_element_type=jnp.float32)
        o_ref[...] = jnp.maximum(acc, 0.0).astype(jnp.bfloat16)

    spec = pltpu.PrefetchScalarGridSpec(
        num_scalar_prefetch=1, grid=(PT // MOE_BT,),
        in_specs=[pl.BlockSpec((MOE_BT, D), lambda t, te: (t, 0)),
                  pl.BlockSpec((1, D, DFF), lambda t, te: (te[t], 0, 0))],
        out_specs=pl.BlockSpec((MOE_BT, DFF), lambda t, te: (t, 0)),
    )
    return pl.pallas_call(
        body, grid_spec=spec,
        out_shape=jax.ShapeDtypeStruct((PT, DFF), jnp.bfloat16))(
            te, xg, wi.astype(jnp.bfloat16))


def _moe_mm2(hm, wo, te):
    PT = hm.shape[0]

    def body(te_ref, x_ref, w_ref, o_ref):
        del te_ref
        o_ref[...] = jax.lax.dot_general(
            x_ref[...], w_ref[0], (((1,), (0,)), ((), ())),
            preferred_element_type=jnp.float32)

    spec = pltpu.PrefetchScalarGridSpec(
        num_scalar_prefetch=1, grid=(PT // MOE_BT,),
        in_specs=[pl.BlockSpec((MOE_BT, DFF), lambda t, te: (t, 0)),
                  pl.BlockSpec((1, DFF, D), lambda t, te: (te[t], 0, 0))],
        out_specs=pl.BlockSpec((MOE_BT, D), lambda t, te: (t, 0)),
    )
    return pl.pallas_call(
        body, grid_spec=spec,
        out_shape=jax.ShapeDtypeStruct((PT, D), jnp.float32))(
            te, hm, wo.astype(jnp.bfloat16))


def _moe_dispatch(lg, T):
    probs = jax.nn.softmax(lg[:, :E], -1)
    topv, topi = jax.lax.top_k(probs, TOPK)
    gates = topv / jnp.sum(topv, -1, keepdims=True)
    F = TOPK * T
    eflat = topi.reshape(F)
    gflat = gates.reshape(F)
    tok = jnp.arange(F, dtype=jnp.int32) // TOPK
    order = jnp.argsort(eflat)
    es = eflat[order]
    ts = tok[order]
    sizes = jnp.bincount(eflat, length=E).astype(jnp.int32)
    off = jnp.concatenate([jnp.zeros(1, jnp.int32), jnp.cumsum(sizes)[:-1]])
    ps = ((sizes + MOE_BT - 1) // MOE_BT) * MOE_BT
    poff = jnp.concatenate([jnp.zeros(1, jnp.int32), jnp.cumsum(ps)[:-1]])
    ppos = jnp.arange(F, dtype=jnp.int32) - off[es] + poff[es]
    PT = F + E * MOE_BT
    tokp = jnp.zeros(PT, jnp.int32).at[ppos].set(ts)
    tiles = jnp.arange(PT // MOE_BT, dtype=jnp.int32) * MOE_BT
    te = (jnp.searchsorted(poff, tiles, side='right') - 1).astype(jnp.int32)
    te = jnp.clip(te, 0, E - 1)
    ipos = jnp.zeros(F, jnp.int32).at[order].set(ppos)
    return tokp, te, ipos, gflat


def _lm_loss(h2, fln, emb_pad, labels2d, bt=1024, bv=2048):
    T = h2.shape[0]
    nv = VP // bv
    scale = float(D) ** -0.5

    def body(h_ref, ln_ref, e_ref, lbl_ref, o_ref, m_s, s_s, l_s):
        vc = pl.program_id(1)
        hv = h_ref[...]
        hn = hv * jax.lax.rsqrt(jnp.mean(hv * hv, 1, keepdims=True) + 1e-6)
        hn = (hn * ln_ref[...] * scale).astype(jnp.bfloat16)
        lg = jax.lax.dot_general(hn, e_ref[...], (((1,), (1,)), ((), ())),
                                 preferred_element_type=jnp.float32)
        cols = vc * bv + jax.lax.broadcasted_iota(jnp.int32, (bt, bv), 1)
        lg = jnp.where(cols < V, lg, NEG)
        lblv = lbl_ref[:, 0:1]
        lhit = jnp.sum(jnp.where(cols == lblv, lg, 0.0), 1, keepdims=True)
        mc = jnp.max(lg, 1, keepdims=True)

        @pl.when(vc == 0)
        def _init():
            m_s[...] = jnp.broadcast_to(mc, (bt, 128))
            s_s[...] = jnp.broadcast_to(
                jnp.sum(jnp.exp(lg - mc), 1, keepdims=True), (bt, 128))
            l_s[...] = jnp.broadcast_to(lhit, (bt, 128))

        @pl.when(vc > 0)
        def _update():
            m_old = jnp.max(m_s[...], 1, keepdims=True)
            s_old = jnp.max(s_s[...], 1, keepdims=True)
            m_new = jnp.maximum(m_old, mc)
            s_new = (s_old * jnp.exp(m_old - m_new)
                     + jnp.sum(jnp.exp(lg - m_new), 1, keepdims=True))
            m_s[...] = jnp.broadcast_to(m_new, (bt, 128))
            s_s[...] = jnp.broadcast_to(s_new, (bt, 128))
            l_s[...] = l_s[...] + lhit

        @pl.when(vc == nv - 1)
        def _finish():
            o_ref[...] = m_s[...] + jnp.log(s_s[...]) - l_s[...]

    return pl.pallas_call(
        body, grid=(T // bt, nv),
        in_specs=[pl.BlockSpec((bt, D), lambda t, v: (t, 0)),
                  pl.BlockSpec((1, D), lambda t, v: (0, 0)),
                  pl.BlockSpec((bv, D), lambda t, v: (v, 0)),
                  pl.BlockSpec((bt, 128), lambda t, v: (t, 0))],
        out_specs=pl.BlockSpec((bt, 128), lambda t, v: (t, 0)),
        out_shape=jax.ShapeDtypeStruct((T, 128), jnp.float32),
        scratch_shapes=[pltpu.VMEM((bt, 128), jnp.float32)] * 3,
    )(h2, fln.reshape(1, D), emb_pad, labels2d)


def _bucket(relpos, bidirectional):
    if bidirectional:
        nb = NB // 2
        rb = (relpos > 0).astype(jnp.int32) * nb
        rp = jnp.abs(relpos)
    else:
        nb = NB
        rb = jnp.zeros_like(relpos)
        rp = -jnp.minimum(relpos, 0)
    max_exact = nb // 2
    is_small = rp < max_exact
    large = max_exact + (jnp.log(rp.astype(jnp.float32) / max_exact + 1e-9)
                         / np.log(MAXD / max_exact)
                         * (nb - max_exact)).astype(jnp.int32)
    large = jnp.minimum(large, nb - 1)
    return rb + jnp.where(is_small, rp, large)


def _mk_bias_win(table, S, bidirectional, bq=256):
    d = jnp.arange(-(S - 1), S)
    diag = table[_bucket(d, bidirectional)].T
    diag = jnp.pad(diag, ((0, 0), (0, 1)))
    QT = S // bq
    SW = S + bq
    wins = [jax.lax.slice(diag, (0, S - (qt + 1) * bq),
                          (H, S - (qt + 1) * bq + SW)) for qt in range(QT)]
    w = jnp.stack(wins, axis=1)
    return w.reshape(H * QT, 1, SW)


def _heads(t, B, S):
    return t.reshape(B, S, H, DK).transpose(0, 2, 1, 3)


def _unheads(t, T):
    return t.transpose(0, 2, 1, 3).reshape(T, INNER)


def kernel(input_ids, emb, enc_rel, dec_rel, enc_ln1, enc_q, enc_k, enc_v,
           enc_o, enc_ln2, enc_wi, enc_wo, enc_fln, dec_ln1, dec_q, dec_k,
           dec_v, dec_o, dec_ln2, dec_cq, dec_ck, dec_cv, dec_co, dec_ln3,
           dec_router, dec_wi, dec_wo, dec_fln):
    B, S = input_ids.shape
    T = B * S
    labels = input_ids.reshape(T)
    dec_ids = jnp.concatenate(
        [jnp.zeros_like(input_ids[:, :1]), input_ids[:, :-1]], 1)
    enc_bw = _mk_bias_win(enc_rel, S, True)
    dec_bw = _mk_bias_win(dec_rel, S, False)

    h = emb[input_ids].reshape(T, D)
    for i in range(L):
        qkv_w = jnp.concatenate([enc_q[i], enc_k[i], enc_v[i]], axis=1)
        qkv = _matmul(h, qkv_w, lnw=enc_ln1[i])
        q, k, v = (qkv[:, :INNER], qkv[:, INNER:2 * INNER], qkv[:, 2 * INNER:])
        ao = _attention(_heads(q, B, S), _heads(k, B, S), _heads(v, B, S),
                        bias_win=enc_bw)
        h = _matmul(_unheads(ao, T), enc_o[i], res=h)
        t1 = _matmul(h, enc_wi[i], lnw=enc_ln2[i], relu=True,
                     out_dtype=jnp.bfloat16)
        h = _matmul(t1, enc_wo[i], res=h)
    h_enc = h

    hd = emb[dec_ids].reshape(T, D)
    for i in range(L):
        qkv_w = jnp.concatenate([dec_q[i], dec_k[i], dec_v[i]], axis=1)
        qkv = _matmul(hd, qkv_w, lnw=dec_ln1[i])
        q, k, v = (qkv[:, :INNER], qkv[:, INNER:2 * INNER], qkv[:, 2 * INNER:])
        ao = _attention(_heads(q, B, S), _heads(k, B, S), _heads(v, B, S),
                        bias_win=dec_bw, causal=True)
        hd = _matmul(_unheads(ao, T), dec_o[i], res=hd)

        cq = _matmul(hd, dec_cq[i], lnw=dec_ln2[i])
        ckv_w = jnp.concatenate([dec_ck[i], dec_cv[i]], axis=1)
        ckv = _matmul(h_enc, ckv_w, lnw=enc_fln)
        ck, cv = ckv[:, :INNER], ckv[:, INNER:]
        co = _attention(_heads(cq, B, S), _heads(ck, B, S), _heads(cv, B, S))
        hd = _matmul(_unheads(co, T), dec_co[i], res=hd)

        rw_pad = jnp.pad(dec_router[i], ((0, 0), (0, 128 - E)))
        xln, lg = _router(hd, dec_ln3[i], rw_pad)
        tokp, te, ipos, gflat = _moe_dispatch(lg, T)
        xg = xln[tokp]
        hm = _moe_mm1(xg, dec_wi[i], te)
        ys = _moe_mm2(hm, dec_wo[i], te)
        y = (ys[ipos] * gflat[:, None]).reshape(T, TOPK, D).sum(1)
        hd = hd + y

    emb_pad = jnp.pad(emb, ((0, VP - V), (0, 0))).astype(jnp.bfloat16)
    labels2d = jnp.broadcast_to(labels[:, None], (T, 128)).astype(jnp.int32)
    nll = _lm_loss(hd, dec_fln, emb_pad, labels2d)
    return jnp.mean(nll[:, 0])


# speedup vs baseline: 7.2156x; 1.1030x over previous
import functools

import jax
import jax.numpy as jnp
import numpy as np
from jax.experimental import pallas as pl
from jax.experimental.pallas import tpu as pltpu

V = 32128; D = 1024; L = 2; H = 8; DK = 64; INNER = H * DK; DFF = 4096
E = 8; TOPK = 2; NB = 32; MAXD = 128
NEG = -1e30
VP = 32768
MOE_BT = 256


def _matmul(x, w, lnw=None, res=None, relu=False, bm=None, bn=512,
            out_dtype=jnp.float32):
    M, K = x.shape
    N = w.shape[1]
    if bm is None:
        bm = 1024 if K <= 1536 else 512
    has_ln = lnw is not None
    has_res = res is not None
    w = w.astype(jnp.bfloat16)

    def body(*refs):
        i = 0
        x_ref = refs[i]; i += 1
        if has_ln:
            ln_ref = refs[i]; i += 1
        w_ref = refs[i]; i += 1
        if has_res:
            r_ref = refs[i]; i += 1
        o_ref = refs[i]
        xv = x_ref[...]
        if has_ln:
            xf = xv.astype(jnp.float32)
            xf = xf * jax.lax.rsqrt(jnp.mean(xf * xf, 1, keepdims=True) + 1e-6)
            xv = xf * ln_ref[...]
        acc = jax.lax.dot_general(xv.astype(jnp.bfloat16), w_ref[...],
                                  (((1,), (0,)), ((), ())),
                                  preferred_element_type=jnp.float32)
        if relu:
            acc = jnp.maximum(acc, 0.0)
        if has_res:
            acc = acc + r_ref[...]
        o_ref[...] = acc.astype(o_ref.dtype)

    in_specs = [pl.BlockSpec((bm, K), lambda i, j: (i, 0))]
    ins = [x]
    if has_ln:
        in_specs.append(pl.BlockSpec((1, K), lambda i, j: (0, 0)))
        ins.append(lnw.reshape(1, K))
    in_specs.append(pl.BlockSpec((K, bn), lambda i, j: (0, j)))
    ins.append(w)
    if has_res:
        in_specs.append(pl.BlockSpec((bm, bn), lambda i, j: (i, j)))
        ins.append(res)
    return pl.pallas_call(
        body, grid=(M // bm, N // bn), in_specs=in_specs,
        out_specs=pl.BlockSpec((bm, bn), lambda i, j: (i, j)),
        out_shape=jax.ShapeDtypeStruct((M, N), out_dtype),
    )(*ins)


def _attention(q, k, v, bias_win=None, causal=False, bq=256):
    B, Hh, S, dk = q.shape
    has_bias = bias_win is not None
    QT = S // bq
    SW = S + bq

    def body(*refs):
        i = 0
        q_ref = refs[i]; i += 1
        k_ref = refs[i]; i += 1
        v_ref = refs[i]; i += 1
        if has_bias:
            b_ref = refs[i]; i += 1
            o_ref = refs[i]; i += 1
            bias_s = refs[i]
        else:
            o_ref = refs[i]
        qv = q_ref[0, 0].astype(jnp.bfloat16)
        kv = k_ref[0, 0].astype(jnp.bfloat16)
        s = jax.lax.dot_general(qv, kv, (((1,), (1,)), ((), ())),
                                preferred_element_type=jnp.float32)
        if has_bias:
            @pl.when(pl.program_id(2) == 0)
            def _mk():
                ri = jax.lax.broadcasted_iota(jnp.int32, (bq, SW), 0)
                srow = (bq - 1) - ri
                X = jnp.broadcast_to(b_ref[0], (bq, SW))
                kk = 1
                while kk < bq:
                    rolled = jnp.concatenate([X[:, kk:], X[:, :kk]], axis=1)
                    X = jnp.where((srow & kk) != 0, rolled, X)
                    kk *= 2
                bias_s[...] = X[:, :S]
            s = s + bias_s[...].astype(jnp.float32)
        if causal:
            qt_id = pl.program_id(1) if has_bias else pl.program_id(2)
            rows = qt_id * bq + jax.lax.broadcasted_iota(
                jnp.int32, (bq, S), 0)
            cols = jax.lax.broadcasted_iota(jnp.int32, (bq, S), 1)
            s = s + jnp.where(cols <= rows, 0.0, -1e9)
        m = jnp.max(s, 1, keepdims=True)
        p = jnp.exp(s - m)
        a = (p / jnp.sum(p, 1, keepdims=True)).astype(jnp.bfloat16)
        o_ref[0, 0] = jax.lax.dot_general(a, v_ref[0, 0].astype(jnp.bfloat16),
                                          (((1,), (0,)), ((), ())),
                                          preferred_element_type=jnp.float32)

    if has_bias:
        in_specs = [
            pl.BlockSpec((1, 1, bq, dk), lambda h, i, b: (b, h, i, 0)),
            pl.BlockSpec((1, 1, S, dk), lambda h, i, b: (b, h, 0, 0)),
            pl.BlockSpec((1, 1, S, dk), lambda h, i, b: (b, h, 0, 0)),
            pl.BlockSpec((1, 1, SW), lambda h, i, b: (h * QT + i, 0, 0)),
        ]
        return pl.pallas_call(
            body, grid=(Hh, S // bq, B), in_specs=in_specs,
            out_specs=pl.BlockSpec((1, 1, bq, dk),
                                   lambda h, i, b: (b, h, i, 0)),
            out_shape=jax.ShapeDtypeStruct((B, Hh, S, dk), jnp.float32),
            scratch_shapes=[pltpu.VMEM((bq, S), jnp.bfloat16)],
        )(q, k, v, bias_win)
    in_specs = [
        pl.BlockSpec((1, 1, bq, dk), lambda b, h, i: (b, h, i, 0)),
        pl.BlockSpec((1, 1, S, dk), lambda b, h, i: (b, h, 0, 0)),
        pl.BlockSpec((1, 1, S, dk), lambda b, h, i: (b, h, 0, 0)),
    ]
    return pl.pallas_call(
        body, grid=(B, Hh, S // bq), in_specs=in_specs,
        out_specs=pl.BlockSpec((1, 1, bq, dk), lambda b, h, i: (b, h, i, 0)),
        out_shape=jax.ShapeDtypeStruct((B, Hh, S, dk), jnp.float32),
    )(q, k, v)


def _router(h2, lnw, rw_pad, bm=512):
    T = h2.shape[0]

    def body(x_ref, ln_ref, rw_ref, xln_ref, lg_ref):
        xv = x_ref[...]
        xn = xv * jax.lax.rsqrt(jnp.mean(xv * xv, 1, keepdims=True) + 1e-6)
        xn = xn * ln_ref[...]
        xln_ref[...] = xn
        lg = jax.lax.dot_general(xn, rw_ref[...], (((1,), (0,)), ((), ())),
                                 preferred_element_type=jnp.float32)
        lanes = jax.lax.broadcasted_iota(jnp.int32, (bm, 128), 1)
        lg_ref[...] = jnp.where(lanes < E, lg, NEG)

    return pl.pallas_call(
        body, grid=(T // bm,),
        in_specs=[pl.BlockSpec((bm, D), lambda i: (i, 0)),
                  pl.BlockSpec((1, D), lambda i: (0, 0)),
                  pl.BlockSpec((D, 128), lambda i: (0, 0))],
        out_specs=(pl.BlockSpec((bm, D), lambda i: (i, 0)),
                   pl.BlockSpec((bm, 128), lambda i: (i, 0))),
        out_shape=(jax.ShapeDtypeStruct((T, D), jnp.float32),
                   jax.ShapeDtypeStruct((T, 128), jnp.float32)),
    )(h2, lnw.reshape(1, D), rw_pad)


def _moe_mm1(xg, wi, te):
    PT = xg.shape[0]

    def body(te_ref, x_ref, w_ref, o_ref):
        del te_ref
        acc = jax.lax.dot_general(x_ref[...].astype(jnp.bfloat16), w_ref[0],
                                  (((1,), (0,)), ((), ())),
                                  preferred_element_type=jnp.float32)
        o_ref[...] = jnp.maximum(acc, 0.0).astype(jnp.bfloat16)

    spec = pltpu.PrefetchScalarGridSpec(
        num_scalar_prefetch=1, grid=(PT // MOE_BT,),
        in_specs=[pl.BlockSpec((MOE_BT, D), lambda t, te: (t, 0)),
                  pl.BlockSpec((1, D, DFF), lambda t, te: (te[t], 0, 0))],
        out_specs=pl.BlockSpec((MOE_BT, DFF), lambda t, te: (t, 0)),
    )
    return pl.pallas_call(
        body, grid_spec=spec,
        out_shape=jax.ShapeDtypeStruct((PT, DFF), jnp.bfloat16))(
            te, xg, wi.astype(jnp.bfloat16))


def _moe_mm2(hm, wo, te):
    PT = hm.shape[0]

    def body(te_ref, x_ref, w_ref, o_ref):
        del te_ref
        o_ref[...] = jax.lax.dot_general(
            x_ref[...], w_ref[0], (((1,), (0,)), ((), ())),
            preferred_element_type=jnp.float32)

    spec = pltpu.PrefetchScalarGridSpec(
        num_scalar_prefetch=1, grid=(PT // MOE_BT,),
        in_specs=[pl.BlockSpec((MOE_BT, DFF), lambda t, te: (t, 0)),
                  pl.BlockSpec((1, DFF, D), lambda t, te: (te[t], 0, 0))],
        out_specs=pl.BlockSpec((MOE_BT, D), lambda t, te: (t, 0)),
    )
    return pl.pallas_call(
        body, grid_spec=spec,
        out_shape=jax.ShapeDtypeStruct((PT, D), jnp.float32))(
            te, hm, wo.astype(jnp.bfloat16))


def _moe_dispatch(lg, T):
    probs = jax.nn.softmax(lg[:, :E], -1)
    topv, topi = jax.lax.top_k(probs, TOPK)
    gates = topv / jnp.sum(topv, -1, keepdims=True)
    F = TOPK * T
    eflat = topi.reshape(F)
    gflat = gates.reshape(F)
    tok = jnp.arange(F, dtype=jnp.int32) // TOPK
    order = jnp.argsort(eflat)
    es = eflat[order]
    ts = tok[order]
    sizes = jnp.bincount(eflat, length=E).astype(jnp.int32)
    off = jnp.concatenate([jnp.zeros(1, jnp.int32), jnp.cumsum(sizes)[:-1]])
    ps = ((sizes + MOE_BT - 1) // MOE_BT) * MOE_BT
    poff = jnp.concatenate([jnp.zeros(1, jnp.int32), jnp.cumsum(ps)[:-1]])
    ppos = jnp.arange(F, dtype=jnp.int32) - off[es] + poff[es]
    PT = F + E * MOE_BT
    tokp = jnp.zeros(PT, jnp.int32).at[ppos].set(ts)
    tiles = jnp.arange(PT // MOE_BT, dtype=jnp.int32) * MOE_BT
    te = (jnp.searchsorted(poff, tiles, side='right') - 1).astype(jnp.int32)
    te = jnp.clip(te, 0, E - 1)
    ipos = jnp.zeros(F, jnp.int32).at[order].set(ppos)
    return tokp, te, ipos, gflat


def _lm_loss(h2, fln, emb_pad, labels2d, bt=1024, bv=2048):
    T = h2.shape[0]
    nv = VP // bv
    scale = float(D) ** -0.5

    def body(h_ref, ln_ref, e_ref, lbl_ref, o_ref, m_s, s_s, l_s):
        vc = pl.program_id(1)
        hv = h_ref[...]
        hn = hv * jax.lax.rsqrt(jnp.mean(hv * hv, 1, keepdims=True) + 1e-6)
        hn = (hn * ln_ref[...] * scale).astype(jnp.bfloat16)
        lg = jax.lax.dot_general(hn, e_ref[...], (((1,), (1,)), ((), ())),
                                 preferred_element_type=jnp.float32)
        cols = vc * bv + jax.lax.broadcasted_iota(jnp.int32, (bt, bv), 1)
        lg = jnp.where(cols < V, lg, NEG)
        lblv = lbl_ref[:, 0:1]
        lhit = jnp.sum(jnp.where(cols == lblv, lg, 0.0), 1, keepdims=True)
        mc = jnp.max(lg, 1, keepdims=True)

        @pl.when(vc == 0)
        def _init():
            m_s[...] = jnp.broadcast_to(mc, (bt, 128))
            s_s[...] = jnp.broadcast_to(
                jnp.sum(jnp.exp(lg - mc), 1, keepdims=True), (bt, 128))
            l_s[...] = jnp.broadcast_to(lhit, (bt, 128))

        @pl.when(vc > 0)
        def _update():
            m_old = jnp.max(m_s[...], 1, keepdims=True)
            s_old = jnp.max(s_s[...], 1, keepdims=True)
            m_new = jnp.maximum(m_old, mc)
            s_new = (s_old * jnp.exp(m_old - m_new)
                     + jnp.sum(jnp.exp(lg - m_new), 1, keepdims=True))
            m_s[...] = jnp.broadcast_to(m_new, (bt, 128))
            s_s[...] = jnp.broadcast_to(s_new, (bt, 128))
            l_s[...] = l_s[...] + lhit

        @pl.when(vc == nv - 1)
        def _finish():
            o_ref[...] = m_s[...] + jnp.log(s_s[...]) - l_s[...]

    return pl.pallas_call(
        body, grid=(T // bt, nv),
        in_specs=[pl.BlockSpec((bt, D), lambda t, v: (t, 0)),
                  pl.BlockSpec((1, D), lambda t, v: (0, 0)),
                  pl.BlockSpec((bv, D), lambda t, v: (v, 0)),
                  pl.BlockSpec((bt, 128), lambda t, v: (t, 0))],
        out_specs=pl.BlockSpec((bt, 128), lambda t, v: (t, 0)),
        out_shape=jax.ShapeDtypeStruct((T, 128), jnp.float32),
        scratch_shapes=[pltpu.VMEM((bt, 128), jnp.float32)] * 3,
    )(h2, fln.reshape(1, D), emb_pad, labels2d)


def _bucket(relpos, bidirectional):
    if bidirectional:
        nb = NB // 2
        rb = (relpos > 0).astype(jnp.int32) * nb
        rp = jnp.abs(relpos)
    else:
        nb = NB
        rb = jnp.zeros_like(relpos)
        rp = -jnp.minimum(relpos, 0)
    max_exact = nb // 2
    is_small = rp < max_exact
    large = max_exact + (jnp.log(rp.astype(jnp.float32) / max_exact + 1e-9)
                         / np.log(MAXD / max_exact)
                         * (nb - max_exact)).astype(jnp.int32)
    large = jnp.minimum(large, nb - 1)
    return rb + jnp.where(is_small, rp, large)


def _mk_bias_win(table, S, bidirectional, bq=256):
    d = jnp.arange(-(S - 1), S)
    diag = table[_bucket(d, bidirectional)].T
    diag = jnp.pad(diag, ((0, 0), (0, 1)))
    QT = S // bq
    SW = S + bq
    wins = [jax.lax.slice(diag, (0, S - (qt + 1) * bq),
                          (H, S - (qt + 1) * bq + SW)) for qt in range(QT)]
    w = jnp.stack(wins, axis=1)
    return w.reshape(H * QT, 1, SW).astype(jnp.bfloat16)


def _heads(t, B, S):
    return t.reshape(B, S, H, DK).transpose(0, 2, 1, 3)


def _unheads(t, T):
    return t.transpose(0, 2, 1, 3).reshape(T, INNER)


def kernel(input_ids, emb, enc_rel, dec_rel, enc_ln1, enc_q, enc_k, enc_v,
           enc_o, enc_ln2, enc_wi, enc_wo, enc_fln, dec_ln1, dec_q, dec_k,
           dec_v, dec_o, dec_ln2, dec_cq, dec_ck, dec_cv, dec_co, dec_ln3,
           dec_router, dec_wi, dec_wo, dec_fln):
    B, S = input_ids.shape
    T = B * S
    labels = input_ids.reshape(T)
    dec_ids = jnp.concatenate(
        [jnp.zeros_like(input_ids[:, :1]), input_ids[:, :-1]], 1)
    enc_bw = _mk_bias_win(enc_rel, S, True)
    dec_bw = _mk_bias_win(dec_rel, S, False)

    h = emb[input_ids].reshape(T, D)
    for i in range(L):
        qkv_w = jnp.concatenate([enc_q[i], enc_k[i], enc_v[i]], axis=1)
        qkv = _matmul(h, qkv_w, lnw=enc_ln1[i])
        q, k, v = (qkv[:, :INNER], qkv[:, INNER:2 * INNER], qkv[:, 2 * INNER:])
        ao = _attention(_heads(q, B, S), _heads(k, B, S), _heads(v, B, S),
                        bias_win=enc_bw)
        h = _matmul(_unheads(ao, T), enc_o[i], res=h)
        t1 = _matmul(h, enc_wi[i], lnw=enc_ln2[i], relu=True,
                     out_dtype=jnp.bfloat16)
        h = _matmul(t1, enc_wo[i], res=h)
    h_enc = h

    hd = emb[dec_ids].reshape(T, D)
    for i in range(L):
        qkv_w = jnp.concatenate([dec_q[i], dec_k[i], dec_v[i]], axis=1)
        qkv = _matmul(hd, qkv_w, lnw=dec_ln1[i])
        q, k, v = (qkv[:, :INNER], qkv[:, INNER:2 * INNER], qkv[:, 2 * INNER:])
        ao = _attention(_heads(q, B, S), _heads(k, B, S), _heads(v, B, S),
                        bias_win=dec_bw, causal=True)
        hd = _matmul(_unheads(ao, T), dec_o[i], res=hd)

        cq = _matmul(hd, dec_cq[i], lnw=dec_ln2[i])
        ckv_w = jnp.concatenate([dec_ck[i], dec_cv[i]], axis=1)
        ckv = _matmul(h_enc, ckv_w, lnw=enc_fln)
        ck, cv = ckv[:, :INNER], ckv[:, INNER:]
        co = _attention(_heads(cq, B, S), _heads(ck, B, S), _heads(cv, B, S))
        hd = _matmul(_unheads(co, T), dec_co[i], res=hd)

        rw_pad = jnp.pad(dec_router[i], ((0, 0), (0, 128 - E)))
        xln, lg = _router(hd, dec_ln3[i], rw_pad)
        tokp, te, ipos, gflat = _moe_dispatch(lg, T)
        xg = xln[tokp]
        hm = _moe_mm1(xg, dec_wi[i], te)
        ys = _moe_mm2(hm, dec_wo[i], te)
        y = (ys[ipos] * gflat[:, None]).reshape(T, TOPK, D).sum(1)
        hd = hd + y

    emb_pad = jnp.pad(emb, ((0, VP - V), (0, 0))).astype(jnp.bfloat16)
    labels2d = jnp.broadcast_to(labels[:, None], (T, 128)).astype(jnp.int32)
    nll = _lm_loss(hd, dec_fln, emb_pad, labels2d)
    return jnp.mean(nll[:, 0])
